# Initial kernel scaffold; baseline (speedup 1.0000x reference)
import jax
import jax.numpy as jnp
from jax.experimental import pallas as pl


def kernel(x, tables):
    raise NotImplementedError("write your pallas kernel here")


# speedup vs baseline: 17.8462x; 17.8462x over previous
import functools

import jax
import jax.numpy as jnp
from jax import lax
from jax.experimental import pallas as pl
from jax.experimental.pallas import tpu as pltpu
from jax.experimental.pallas import tpu_sc as plsc

B = 16384
F = 100
V = 1000
D = 64
R = 40

NC = 2
NS = 16
NW = NC * NS
BPW = B // NW
GB = 128
NBLK = BPW // GB
LANES = 16
SEGS = GB * D // LANES


@functools.lru_cache(maxsize=1)
def _make_sc_sum():
    mesh = plsc.VectorSubcoreMesh(
        core_axis_name="c", subcore_axis_name="s", num_cores=NC, num_subcores=NS
    )

    @functools.partial(
        pl.kernel,
        out_type=jax.ShapeDtypeStruct((B, D), jnp.float32),
        mesh=mesh,
        scratch_types=[
            pltpu.VMEM((GB,), jnp.int32),
            pltpu.VMEM((GB,), jnp.int32),
            pltpu.VMEM((GB, D), jnp.float32),
            pltpu.VMEM((GB, D), jnp.float32),
            pltpu.SemaphoreType.DMA,
        ],
        compiler_params=pltpu.CompilerParams(use_tc_tiling_on_sc=False),
    )
    def _sc_sum(xt_hbm, tab_hbm, sums_hbm, idxraw_v, idx_v, rows_v, acc_v, sem):
        wid = lax.axis_index("s") * NC + lax.axis_index("c")

        def blk_body(blk):
            base = wid * BPW + blk * GB

            zero = jnp.zeros((LANES,), jnp.float32)
            for r in range(GB):
                for c in range(D // LANES):
                    acc_v[r, pl.ds(c * LANES, LANES)] = zero

            def feat_body(j):
                pltpu.sync_copy(xt_hbm.at[j, pl.ds(base, GB)], idxraw_v)
                off = j * V
                for t in range(GB // LANES):
                    sl = pl.ds(t * LANES, LANES)
                    idx_v[sl] = idxraw_v[sl] + off
                pltpu.async_copy(tab_hbm.at[idx_v], rows_v, sem).wait()
                for r in range(GB):
                    for c in range(D // LANES):
                        sl = pl.ds(c * LANES, LANES)
                        acc_v[r, sl] = acc_v[r, sl] + rows_v[r, sl]

            pl.loop(0, F)(feat_body)
            pltpu.sync_copy(acc_v, sums_hbm.at[pl.ds(base, GB)])

        pl.loop(0, NBLK)(blk_body)

    return _sc_sum


def _bcast_body(s_ref, o_ref):
    o_ref[...] = jnp.broadcast_to(s_ref[...][:, None, :], o_ref.shape)


_BM = 256


def _bcast(sums):
    return pl.pallas_call(
        _bcast_body,
        grid=(B // _BM,),
        in_specs=[pl.BlockSpec((_BM, D), lambda i: (i, 0))],
        out_specs=pl.BlockSpec((_BM, R, D), lambda i: (i, 0, 0)),
        out_shape=jax.ShapeDtypeStruct((B, R, D), jnp.float32),
    )(sums)


def kernel(x, tables):
    xt = x.T
    tab = tables.reshape(F * V, D)
    sums = _make_sc_sum()(xt, tab)
    return _bcast(sums)


# speedup vs baseline: 55.1602x; 3.0909x over previous
import functools

import jax
import jax.numpy as jnp
from jax import lax
from jax.experimental import pallas as pl
from jax.experimental.pallas import tpu as pltpu
from jax.experimental.pallas import tpu_sc as plsc

B = 16384
F = 100
V = 1000
D = 64
R = 40

NC = 2
NS = 16
NW = NC * NS
BPW = B // NW
GB = 128
NBLK = BPW // GB
LANES = 16
CS = D // LANES
FG = 5
NG = F // FG


@functools.lru_cache(maxsize=1)
def _make_sc_sum():
    mesh = plsc.VectorSubcoreMesh(
        core_axis_name="c", subcore_axis_name="s", num_cores=NC, num_subcores=NS
    )

    @functools.partial(
        pl.kernel,
        out_type=jax.ShapeDtypeStruct((B, D), jnp.float32),
        mesh=mesh,
        scratch_types=[
            pltpu.VMEM((F, GB), jnp.int32),
            pltpu.VMEM((FG, GB, D), jnp.float32),
            pltpu.VMEM((FG, GB, D), jnp.float32),
            pltpu.VMEM((GB, D), jnp.float32),
            pltpu.SemaphoreType.DMA,
            pltpu.SemaphoreType.DMA,
        ],
        compiler_params=pltpu.CompilerParams(use_tc_tiling_on_sc=False),
    )
    def _sc_sum(xt_hbm, tab_hbm, sums_hbm, idx_v, buf_a, buf_b, acc_v, sem_a, sem_b):
        wid = lax.axis_index("s") * NC + lax.axis_index("c")

        def fire(g, buf, sem):
            for k in range(FG):
                pltpu.async_copy(
                    tab_hbm.at[idx_v.at[g * FG + k]], buf.at[k], sem
                )

        def drain(buf, sem):
            for k in range(FG):
                pltpu.make_async_copy(
                    tab_hbm.at[idx_v.at[k]], buf.at[k], sem
                ).wait()

        def accum(buf):
            def row_body(r):
                for c in range(CS):
                    sl = pl.ds(c * LANES, LANES)
                    s = buf[0, r, sl]
                    for k in range(1, FG):
                        s = s + buf[k, r, sl]
                    acc_v[r, sl] = acc_v[r, sl] + s

            pl.loop(0, GB)(row_body)

        def blk_body(blk):
            base = wid * BPW + blk * GB
            pltpu.sync_copy(xt_hbm.at[:, pl.ds(base, GB)], idx_v)

            def off_body(j):
                off = j * V
                for t in range(GB // LANES):
                    sl = pl.ds(t * LANES, LANES)
                    idx_v[j, sl] = idx_v[j, sl] + off

            pl.loop(0, F)(off_body)

            zero = jnp.zeros((LANES,), jnp.float32)

            def zero_body(r):
                for c in range(CS):
                    acc_v[r, pl.ds(c * LANES, LANES)] = zero

            pl.loop(0, GB)(zero_body)

            fire(0, buf_a, sem_a)
            fire(1, buf_b, sem_b)

            def pair_body(g):
                drain(buf_a, sem_a)
                accum(buf_a)

                @pl.when(g + 2 < NG)
                def _():
                    fire(g + 2, buf_a, sem_a)

                drain(buf_b, sem_b)
                accum(buf_b)

                @pl.when(g + 3 < NG)
                def _():
                    fire(g + 3, buf_b, sem_b)

            pl.loop(0, NG, step=2)(pair_body)
            pltpu.sync_copy(acc_v, sums_hbm.at[pl.ds(base, GB)])

        pl.loop(0, NBLK)(blk_body)

    return _sc_sum


def _bcast_body(s_ref, o_ref):
    o_ref[...] = jnp.broadcast_to(s_ref[...][:, None, :], o_ref.shape)


_BM = 256


def _bcast(sums):
    return pl.pallas_call(
        _bcast_body,
        grid=(B // _BM,),
        in_specs=[pl.BlockSpec((_BM, D), lambda i: (i, 0))],
        out_specs=pl.BlockSpec((_BM, R, D), lambda i: (i, 0, 0)),
        out_shape=jax.ShapeDtypeStruct((B, R, D), jnp.float32),
    )(sums)


def kernel(x, tables):
    xt = x.T
    tab = tables.reshape(F * V, D)
    sums = _make_sc_sum()(xt, tab)
    return _bcast(sums)


# speedup vs baseline: 99.0174x; 1.7951x over previous
import functools

import jax
import jax.numpy as jnp
from jax import lax
from jax.experimental import pallas as pl
from jax.experimental.pallas import tpu as pltpu
from jax.experimental.pallas import tpu_sc as plsc

B = 16384
F = 100
V = 1000
D = 64
R = 40

NC = 2
NS = 16
NW = NC * NS
BPW = B // NW
GB = 128
NBLK = BPW // GB
LANES = 16
CS = D // LANES
FG = 5
NG = F // FG


@functools.lru_cache(maxsize=1)
def _make_sc_sum():
    mesh = plsc.VectorSubcoreMesh(
        core_axis_name="c", subcore_axis_name="s", num_cores=NC, num_subcores=NS
    )

    @functools.partial(
        pl.kernel,
        out_type=jax.ShapeDtypeStruct((B, D), jnp.float32),
        mesh=mesh,
        scratch_types=[
            pltpu.VMEM((F, GB), jnp.int32),
            pltpu.VMEM((FG, GB, D), jnp.float32),
            pltpu.VMEM((FG, GB, D), jnp.float32),
            pltpu.VMEM((GB, D), jnp.float32),
            pltpu.SemaphoreType.DMA,
            pltpu.SemaphoreType.DMA,
        ],
        compiler_params=pltpu.CompilerParams(use_tc_tiling_on_sc=False),
    )
    def _sc_sum(xt_hbm, tab_hbm, sums_hbm, idx_v, buf_a, buf_b, acc_v, sem_a, sem_b):
        wid = lax.axis_index("s") * NC + lax.axis_index("c")

        def fire(g, buf, sem):
            for k in range(FG):
                pltpu.async_copy(
                    tab_hbm.at[idx_v.at[g * FG + k]], buf.at[k], sem
                )

        def drain(buf, sem):
            for k in range(FG):
                pltpu.make_async_copy(
                    tab_hbm.at[idx_v.at[k]], buf.at[k], sem
                ).wait()

        def accum(buf):
            def row_body(r):
                for c in range(CS):
                    sl = pl.ds(c * LANES, LANES)
                    s = buf[0, r, sl]
                    for k in range(1, FG):
                        s = s + buf[k, r, sl]
                    acc_v[r, sl] = acc_v[r, sl] + s

            pl.loop(0, GB)(row_body)

        def blk_body(blk):
            base = wid * BPW + blk * GB
            pltpu.sync_copy(xt_hbm.at[:, pl.ds(base, GB)], idx_v)

            def off_body(j):
                off = j * V
                for t in range(GB // LANES):
                    sl = pl.ds(t * LANES, LANES)
                    idx_v[j, sl] = idx_v[j, sl] + off

            pl.loop(0, F)(off_body)

            zero = jnp.zeros((LANES,), jnp.float32)

            def zero_body(r):
                for c in range(CS):
                    acc_v[r, pl.ds(c * LANES, LANES)] = zero

            pl.loop(0, GB)(zero_body)

            fire(0, buf_a, sem_a)
            fire(1, buf_b, sem_b)

            def pair_body(g):
                drain(buf_a, sem_a)
                accum(buf_a)

                @pl.when(g + 2 < NG)
                def _():
                    fire(g + 2, buf_a, sem_a)

                drain(buf_b, sem_b)
                accum(buf_b)

                @pl.when(g + 3 < NG)
                def _():
                    fire(g + 3, buf_b, sem_b)

            pl.loop(0, NG, step=2)(pair_body)
            pltpu.sync_copy(acc_v, sums_hbm.at[pl.ds(base, GB)])

        pl.loop(0, NBLK)(blk_body)

    return _sc_sum


def _bcast_body(s_ref, o_ref):
    st = s_ref[...].T
    o_ref[...] = jnp.broadcast_to(st[None, :, :], o_ref.shape)


_BM = 512


def _bcast(sums):
    out3 = pl.pallas_call(
        _bcast_body,
        grid=(B // _BM,),
        in_specs=[pl.BlockSpec((_BM, D), lambda i: (i, 0))],
        out_specs=pl.BlockSpec((R, D, _BM), lambda i: (0, 0, i)),
        out_shape=jax.ShapeDtypeStruct((R, D, B), jnp.float32),
    )(sums)
    return jnp.transpose(out3, (2, 0, 1))


def kernel(x, tables):
    xt = x.T
    tab = tables.reshape(F * V, D)
    sums = _make_sc_sum()(xt, tab)
    return _bcast(sums)
